# baseline (speedup 1.0000x reference)
import functools

import jax
import jax.numpy as jnp
from jax import lax
from jax.experimental import pallas as pl
from jax.experimental.pallas import tpu as pltpu
from jax.experimental.pallas import tpu_sc as plsc

NN = 6884
NR = 990
DD = 128
HID = 64
NE = 110144
NC, NS, LL = 2, 16, 16
NWK = NC * NS
EP = 110592
NNP = 6912
TBL = NN * NR
QS = 1703808


def _t1_body(emb_ref, w_ref, b_ref, xb2_ref, y1_ref):
    acc = jnp.dot(emb_ref[...], w_ref[...], preferred_element_type=jnp.float32)
    xb2_ref[...] = acc[:, :DD]
    y1_ref[...] = acc[:, DD:] + b_ref[...]


def _t1(emb, w1, bias1):
    return pl.pallas_call(
        _t1_body,
        out_shape=(
            jax.ShapeDtypeStruct((NN, DD), jnp.float32),
            jax.ShapeDtypeStruct((NN, HID), jnp.float32),
        ),
    )(emb, w1, bias1)


def _t2_body(agg_ref, y1_ref, root2_ref, bias2_ref, x1_ref, y2_ref):
    agg = agg_ref[0, :NN, :] + agg_ref[1, :NN, :]
    x1 = jnp.maximum(agg + y1_ref[...], 0.0)
    x1_ref[...] = x1
    y2_ref[...] = (
        jnp.dot(x1, root2_ref[...], preferred_element_type=jnp.float32)
        + bias2_ref[...]
    )


def _t2(agg1p, y1, root2, bias2):
    return pl.pallas_call(
        _t2_body,
        out_shape=(
            jax.ShapeDtypeStruct((NN, HID), jnp.float32),
            jax.ShapeDtypeStruct((NN, DD), jnp.float32),
        ),
    )(agg1p, y1, root2, bias2)


def _t3_body(agg_ref, y2_ref, x2_ref):
    agg = agg_ref[0, :NN, :] + agg_ref[1, :NN, :]
    x2_ref[...] = jnp.maximum(agg + y2_ref[...], 0.0)


def _t3(agg2p, y2):
    return pl.pallas_call(
        _t3_body,
        out_shape=jax.ShapeDtypeStruct((NN, DD), jnp.float32),
    )(agg2p, y2)


def _t4_body(psq_ref, nsq_ref, out_ref):
    pos = jnp.sqrt(psq_ref[...])
    neg = jnp.sqrt(nsq_ref[...])
    term = jnp.maximum(pos - neg + 1.0, 0.0)
    valid = (lax.iota(jnp.int32, EP) < NE).astype(jnp.float32)
    out_ref[...] = (jnp.sum(term * valid) * (1.0 / NE)).reshape(1, 1)


def _t4(psq, nsq):
    out = pl.pallas_call(
        _t4_body,
        out_shape=jax.ShapeDtypeStruct((1, 1), jnp.float32),
    )(psq, nsq)
    return out[0, 0]


def _counts_jnp(dstp, tp):
    key = dstp * NR + tp
    counts = jnp.zeros((4 * QS,), jnp.float32).at[key].add(1.0)
    c = counts[key]
    half = (key < 2 * QS).astype(jnp.float32)
    return jnp.stack([c * half, c * (1.0 - half)])


def _l1_edges_jnp(xb2, comp, srcp, dstp, tp, norm):
    c = comp[tp]
    msg = (c[:, :1] * xb2[srcp, :HID] + c[:, 1:] * xb2[srcp, HID:]) * norm[:, None]
    agg = jnp.zeros((2, NNP, HID), jnp.float32).at[0, dstp].add(msg)
    return agg


def _l2_edges_jnp(x1, W2f, srcp, dstp, tp, norm):
    x1r = x1.reshape(NN, 4, 16)
    W2 = W2f.reshape(NR, 4, 16, 32)
    msg = jnp.einsum('eci,ecio->eco', x1r[srcp], W2[tp]).reshape(EP, DD)
    msg = msg * norm[:, None]
    agg = jnp.zeros((2, NNP, DD), jnp.float32).at[0, dstp].add(msg)
    return agg


def _score_jnp(x2, rel, ei0p, ei1p, n0p, n1p, etp):
    rv = rel[etp]
    dp = x2[ei0p] + rv - x2[ei1p]
    dn = x2[n0p] + rv - x2[n1p]
    return jnp.sum(dp * dp, axis=1), jnp.sum(dn * dn, axis=1)


def _pad_e(a, fill):
    return jnp.concatenate(
        [a, jnp.full((EP - NE,), fill, a.dtype)], axis=0)


def kernel(emb, comp, bases, root1, bias1, W2, root2, bias2, rel,
           total_index, total_type, edge_index, edge_type):
    w1 = jnp.concatenate(
        [jnp.concatenate([bases[0], bases[1]], axis=1), root1], axis=1)
    srcp = _pad_e(total_index[0], 0)
    dstp = _pad_e(total_index[1], NN)
    tp = _pad_e(total_type, 0)

    xb2, y1 = _t1(emb, w1, bias1)

    cnt2 = _counts_jnp(dstp, tp)
    valid = (lax.iota(jnp.int32, EP) < NE).astype(jnp.float32)
    norm = valid / (cnt2[0] + cnt2[1])

    agg1p = _l1_edges_jnp(xb2, comp, srcp, dstp, tp, norm)
    x1, y2 = _t2(agg1p, y1, root2, bias2)

    agg2p = _l2_edges_jnp(x1, W2.reshape(NR, 4 * 16 * 32), srcp, dstp, tp, norm)
    x2 = _t3(agg2p, y2)

    neg_ei = jax.random.randint(jax.random.key(123), (2, NE), 0, NN)
    ei0p, ei1p = _pad_e(edge_index[0], 0), _pad_e(edge_index[1], 0)
    n0p, n1p = _pad_e(neg_ei[0], 0), _pad_e(neg_ei[1], 0)
    etp = _pad_e(edge_type, 0)
    psq, nsq = _score_jnp(x2, rel, ei0p, ei1p, n0p, n1p, etp)

    return _t4(psq, nsq)


# speedup vs baseline: 13.5182x; 13.5182x over previous
import functools

import jax
import jax.numpy as jnp
from jax import lax
from jax.experimental import pallas as pl
from jax.experimental.pallas import tpu as pltpu
from jax.experimental.pallas import tpu_sc as plsc

NN = 6884
NR = 990
DD = 128
HID = 64
NE = 110144
NC, NS, LL = 2, 16, 16
NWK = NC * NS
EP = 131072
NNP = 6912
TBL = NN * NR
QS = 1703808
QST = QS + 128


def _t1_body(emb_ref, w_ref, b_ref, xb2_ref, y1_ref):
    acc = jnp.dot(emb_ref[...], w_ref[...], preferred_element_type=jnp.float32)
    xb2_ref[...] = acc[:, :DD]
    y1_ref[...] = acc[:, DD:] + b_ref[...]


def _t1(emb, w1, bias1):
    return pl.pallas_call(
        _t1_body,
        out_shape=(
            jax.ShapeDtypeStruct((NN, DD), jnp.float32),
            jax.ShapeDtypeStruct((NN, HID), jnp.float32),
        ),
    )(emb, w1, bias1)


def _t2_body(agg_ref, y1_ref, root2_ref, bias2_ref, x1_ref, y2_ref):
    agg = agg_ref[0, :NN, :HID] + agg_ref[1, :NN, :HID]
    x1 = jnp.maximum(agg + y1_ref[...], 0.0)
    x1_ref[...] = jnp.concatenate(
        [x1, jnp.zeros((NN, DD - HID), jnp.float32)], axis=1)
    y2_ref[...] = (
        jnp.dot(x1, root2_ref[...], preferred_element_type=jnp.float32)
        + bias2_ref[...]
    )


def _t2(agg1p, y1, root2, bias2):
    return pl.pallas_call(
        _t2_body,
        out_shape=(
            jax.ShapeDtypeStruct((NN, DD), jnp.float32),
            jax.ShapeDtypeStruct((NN, DD), jnp.float32),
        ),
    )(agg1p, y1, root2, bias2)


def _t3_body(agg_ref, y2_ref, x2_ref):
    agg = agg_ref[0, :NN, :] + agg_ref[1, :NN, :]
    x2_ref[...] = jnp.maximum(agg + y2_ref[...], 0.0)


def _t3(agg2p, y2):
    return pl.pallas_call(
        _t3_body,
        out_shape=jax.ShapeDtypeStruct((NN, DD), jnp.float32),
    )(agg2p, y2)


_T4B = 1024


def _t4_body(dp_ref, dn_ref, out_ref):
    g = pl.program_id(0)
    dp = dp_ref[...]
    dn = dn_ref[...]
    pos = jnp.sqrt(jnp.sum(dp * dp, axis=1, keepdims=True))
    neg = jnp.sqrt(jnp.sum(dn * dn, axis=1, keepdims=True))
    term = jnp.maximum(pos - neg + 1.0, 0.0)
    row = g * _T4B + lax.broadcasted_iota(jnp.int32, (_T4B, 1), 0)
    term = jnp.where(row < NE, term, 0.0)
    part = (jnp.sum(term) * (1.0 / NE)).reshape(1, 1)

    @pl.when(g == 0)
    def _():
        out_ref[...] = jnp.zeros((1, 1), jnp.float32)

    out_ref[...] += part


def _t4(dp, dn):
    out = pl.pallas_call(
        _t4_body,
        grid=(EP // _T4B,),
        in_specs=[
            pl.BlockSpec((_T4B, DD), lambda g: (g, 0)),
            pl.BlockSpec((_T4B, DD), lambda g: (g, 0)),
        ],
        out_specs=pl.BlockSpec((1, 1), lambda g: (0, 0)),
        out_shape=jax.ShapeDtypeStruct((1, 1), jnp.float32),
    )(dp, dn)
    return out[0, 0]


_MESH_CACHE = []


def _mesh():
    if not _MESH_CACHE:
        _MESH_CACHE.append(plsc.VectorSubcoreMesh(
            core_axis_name="c", subcore_axis_name="s",
            num_cores=NC, num_subcores=NS))
    return _MESH_CACHE[0]
ER = EP // 128
ROWS32 = ER // NWK
ROWS16 = ER // NS


def _iota16():
    return lax.broadcasted_iota(jnp.int32, (LL,), 0)


def _s4_body(x2_hbm, rel_hbm, i0_hbm, i1_hbm, j0_hbm, j1_hbm, et_hbm,
             dp_hbm, dn_hbm,
             i0b, i1b, j0b, j1b, etb, s0b, t0b, u0b, u1b, rvb, sem):
    cid = lax.axis_index("c")
    sid = lax.axis_index("s")
    wid = sid * NC + cid
    row0 = wid * ROWS32
    pltpu.sync_copy(i0_hbm.at[pl.ds(row0, ROWS32)], i0b)
    pltpu.sync_copy(i1_hbm.at[pl.ds(row0, ROWS32)], i1b)
    pltpu.sync_copy(j0_hbm.at[pl.ds(row0, ROWS32)], j0b)
    pltpu.sync_copy(j1_hbm.at[pl.ds(row0, ROWS32)], j1b)
    pltpu.sync_copy(et_hbm.at[pl.ds(row0, ROWS32)], etb)

    def batch(g, _):
        cps = [
            pltpu.async_copy(x2_hbm.at[i0b.at[g]], s0b, sem),
            pltpu.async_copy(x2_hbm.at[i1b.at[g]], t0b, sem),
            pltpu.async_copy(x2_hbm.at[j0b.at[g]], u0b, sem),
            pltpu.async_copy(x2_hbm.at[j1b.at[g]], u1b, sem),
            pltpu.async_copy(rel_hbm.at[etb.at[g]], rvb, sem),
        ]
        for cp in cps:
            cp.wait()

        def edge(e, _c):
            for k in range(8):
                c = pl.ds(k * LL, LL)
                rv = rvb[e, c]
                sv = s0b[e, c]
                uv = u0b[e, c]
                s0b[e, c] = sv + rv - t0b[e, c]
                u0b[e, c] = uv + rv - u1b[e, c]
            return _c

        lax.fori_loop(0, 128, edge, 0)
        ebase = row0 * 128 + g * 128
        pltpu.sync_copy(s0b, dp_hbm.at[pl.ds(ebase, 128)])
        pltpu.sync_copy(u0b, dn_hbm.at[pl.ds(ebase, 128)])
        return 0

    lax.fori_loop(0, ROWS32, batch, 0)


def _s4(x2, rel, i0r, i1r, j0r, j1r, etr):
    f = pl.kernel(
        _s4_body,
        out_type=(
            jax.ShapeDtypeStruct((EP, DD), jnp.float32),
            jax.ShapeDtypeStruct((EP, DD), jnp.float32),
        ),
        mesh=_mesh(),
        scratch_types=[
            pltpu.VMEM((ROWS32, 128), jnp.int32),
            pltpu.VMEM((ROWS32, 128), jnp.int32),
            pltpu.VMEM((ROWS32, 128), jnp.int32),
            pltpu.VMEM((ROWS32, 128), jnp.int32),
            pltpu.VMEM((ROWS32, 128), jnp.int32),
            pltpu.VMEM((128, DD), jnp.float32),
            pltpu.VMEM((128, DD), jnp.float32),
            pltpu.VMEM((128, DD), jnp.float32),
            pltpu.VMEM((128, DD), jnp.float32),
            pltpu.VMEM((128, DD), jnp.float32),
            pltpu.SemaphoreType.DMA,
        ],
    )
    return f(x2, rel, i0r, i1r, j0r, j1r, etr)


def _s1_body(dst_hbm, t_hbm, cnt_hbm,
             tbl, dstb, tb, idxrow, accb, gb, onesb, zbuf, sem):
    cid = lax.axis_index("c")
    sid = lax.axis_index("s")
    row0 = sid * ROWS16

    def of(i, c):
        onesb[pl.ds(i * LL, LL)] = jnp.ones((LL,), jnp.float32)
        return c

    lax.fori_loop(0, 128 // LL, of, 0)

    def zf(i, c):
        zbuf[pl.ds(i * LL, LL)] = jnp.zeros((LL,), jnp.float32)
        return c

    lax.fori_loop(0, 4096 // LL, zf, 0)

    def build_row(g, lo):
        for k in range(8):
            s = pl.ds(k * LL, LL)
            key = dstb[g, s] * NR + tb[g, s]
            m = (key >= lo) & (key < lo + QS)
            idxrow[s] = jnp.where(m, key - lo, QS)

    for q in range(2):
        lo = (2 * cid + q) * QS

        def zt(i, c):
            pltpu.sync_copy(zbuf, tbl.at[pl.ds(sid * (QST // NS) + i * 4096,
                                               4096)])
            return c

        lax.fori_loop(0, QST // NS // 4096, zt, 0)
        plsc.subcore_barrier()

        def scat_sb(sb, c):
            pltpu.sync_copy(dst_hbm.at[pl.ds(row0 + sb * 16, 16)], dstb)
            pltpu.sync_copy(t_hbm.at[pl.ds(row0 + sb * 16, 16)], tb)

            def scat(g, c2):
                build_row(g, lo)
                pltpu.sync_copy(onesb, tbl.at[idxrow], add=True)
                return c2

            lax.fori_loop(0, 16, scat, 0)
            return c

        lax.fori_loop(0, ROWS16 // 16, scat_sb, 0)
        plsc.subcore_barrier()

        def gath_sb(sb, c):
            pltpu.sync_copy(dst_hbm.at[pl.ds(row0 + sb * 16, 16)], dstb)
            pltpu.sync_copy(t_hbm.at[pl.ds(row0 + sb * 16, 16)], tb)

            def gath(g, c2):
                build_row(g, lo)
                pltpu.async_copy(tbl.at[idxrow], gb, sem).wait()
                for k in range(8):
                    s = pl.ds(k * LL, LL)
                    m = idxrow[s] < QS
                    v = jnp.where(m, gb[s], 0.0)
                    if q == 0:
                        accb[sb * 16 + g, s] = v
                    else:
                        accb[sb * 16 + g, s] = accb[sb * 16 + g, s] + v
                return c2

            lax.fori_loop(0, 16, gath, 0)
            return c

        lax.fori_loop(0, ROWS16 // 16, gath_sb, 0)
        if q == 0:
            plsc.subcore_barrier()

    pltpu.sync_copy(accb, cnt_hbm.at[cid, pl.ds(row0, ROWS16)])


def _s1(dstr, tr):
    f = pl.kernel(
        _s1_body,
        out_type=jax.ShapeDtypeStruct((2, ER, 128), jnp.float32),
        mesh=_mesh(),
        scratch_types=[
            pltpu.VMEM_SHARED((QST,), jnp.float32),
            pltpu.VMEM((16, 128), jnp.int32),
            pltpu.VMEM((16, 128), jnp.int32),
            pltpu.VMEM((128,), jnp.int32),
            pltpu.VMEM((ROWS16, 128), jnp.float32),
            pltpu.VMEM((128,), jnp.float32),
            pltpu.VMEM((128,), jnp.float32),
            pltpu.VMEM((4096,), jnp.float32),
            pltpu.SemaphoreType.DMA,
        ],
    )
    return f(dstr, tr)


NTR = NNP // NS


def _s2_body(xb2_hbm, compf_hbm, srcr_hbm, dstr_hbm, tf_hbm, cntf_hbm,
             agg_hbm,
             agg, srcb, dstb, tbf, c0f, c1f, normf, compb, xbuf, msgb,
             zb, sem):
    cid = lax.axis_index("c")
    sid = lax.axis_index("s")
    wid = sid * NC + cid
    row0 = wid * ROWS32
    e0 = wid * (EP // NWK)
    nch = EP // NWK

    pltpu.sync_copy(srcr_hbm.at[pl.ds(row0, ROWS32)], srcb)
    pltpu.sync_copy(dstr_hbm.at[pl.ds(row0, ROWS32)], dstb)
    pltpu.sync_copy(tf_hbm.at[pl.ds(e0, nch)], tbf.at[pl.ds(0, nch)])
    pltpu.sync_copy(cntf_hbm.at[0, pl.ds(e0, nch)], c0f.at[pl.ds(0, nch)])
    pltpu.sync_copy(cntf_hbm.at[1, pl.ds(e0, nch)], c1f.at[pl.ds(0, nch)])
    pltpu.sync_copy(compf_hbm, compb)

    def nrm(i, c):
        s = pl.ds(i * LL, LL)
        ge = e0 + i * LL + _iota16()
        v = jnp.where(ge < NE, 1.0, 0.0)
        normf[s] = v / (c0f[s] + c1f[s])
        return c

    lax.fori_loop(0, nch // LL, nrm, 0)

    def zv(i, c):
        for j in range(128 // LL):
            zb[i, pl.ds(j * LL, LL)] = jnp.zeros((LL,), jnp.float32)
        return c

    lax.fori_loop(0, 54, zv, 0)

    def zt(i, c):
        pltpu.sync_copy(zb, agg.at[pl.ds(sid * NTR + i * 54, 54)])
        return c

    lax.fori_loop(0, NTR // 54, zt, 0)
    plsc.subcore_barrier()

    def zm(e, c):
        for j in range(HID // LL):
            msgb[e, pl.ds(HID + j * LL, LL)] = jnp.zeros((LL,), jnp.float32)
        return c

    lax.fori_loop(0, 128, zm, 0)

    def batch(g, _):
        pltpu.async_copy(xb2_hbm.at[srcb.at[g]], xbuf, sem).wait()

        def edge(e, c2):
            eg = g * 128 + e
            te = tbf[pl.ds(eg, LL)][0]
            cv = compb[pl.ds(te * 2, LL)]
            n = normf[pl.ds(eg, LL)][0]
            a0 = cv[0] * n
            a1 = cv[1] * n
            for j in range(HID // LL):
                s = pl.ds(j * LL, LL)
                s2 = pl.ds(HID + j * LL, LL)
                msgb[e, s] = xbuf[e, s] * a0 + xbuf[e, s2] * a1
            return c2

        lax.fori_loop(0, 128, edge, 0)
        pltpu.sync_copy(msgb, agg.at[dstb.at[g]], add=True)
        return 0

    lax.fori_loop(0, ROWS32, batch, 0)
    plsc.subcore_barrier()
    pltpu.sync_copy(agg.at[pl.ds(sid * NTR, NTR)],
                    agg_hbm.at[cid, pl.ds(sid * NTR, NTR)])


def _s2(xb2, compf, srcr, dstr, tf, cntf):
    f = pl.kernel(
        _s2_body,
        out_type=jax.ShapeDtypeStruct((2, NNP, 128), jnp.float32),
        mesh=_mesh(),
        scratch_types=[
            pltpu.VMEM_SHARED((NNP, 128), jnp.float32),
            pltpu.VMEM((ROWS32, 128), jnp.int32),
            pltpu.VMEM((ROWS32, 128), jnp.int32),
            pltpu.VMEM((EP // NWK + LL,), jnp.int32),
            pltpu.VMEM((EP // NWK + LL,), jnp.float32),
            pltpu.VMEM((EP // NWK + LL,), jnp.float32),
            pltpu.VMEM((EP // NWK + LL,), jnp.float32),
            pltpu.VMEM((2048,), jnp.float32),
            pltpu.VMEM((128, DD), jnp.float32),
            pltpu.VMEM((128, 128), jnp.float32),
            pltpu.VMEM((54, 128), jnp.float32),
            pltpu.SemaphoreType.DMA,
        ],
    )
    return f(xb2, compf, srcr, dstr, tf, cntf)


WB = 8


def _s3_body(x1_hbm, w2_hbm, srcr_hbm, dstr_hbm, tf_hbm, cntf_hbm,
             agg_hbm,
             agg, srcb, dstb, tbf, c0f, c1f, xbuf, msgb, w2b, zb, sem):
    cid = lax.axis_index("c")
    sid = lax.axis_index("s")
    wid = sid * NC + cid
    row0 = wid * ROWS32
    e0 = wid * (EP // NWK)
    nch = EP // NWK

    pltpu.sync_copy(srcr_hbm.at[pl.ds(row0, ROWS32)], srcb)
    pltpu.sync_copy(dstr_hbm.at[pl.ds(row0, ROWS32)], dstb)
    pltpu.sync_copy(tf_hbm.at[pl.ds(e0, nch)], tbf.at[pl.ds(0, nch)])
    pltpu.sync_copy(cntf_hbm.at[0, pl.ds(e0, nch)], c0f.at[pl.ds(0, nch)])
    pltpu.sync_copy(cntf_hbm.at[1, pl.ds(e0, nch)], c1f.at[pl.ds(0, nch)])

    def nrm(i, c):
        s = pl.ds(i * LL, LL)
        ge = e0 + i * LL + _iota16()
        v = jnp.where(ge < NE, 1.0, 0.0)
        c0f[s] = v / (c0f[s] + c1f[s])
        return c

    lax.fori_loop(0, nch // LL, nrm, 0)

    def zv(i, c):
        for j in range(128 // LL):
            zb[i, pl.ds(j * LL, LL)] = jnp.zeros((LL,), jnp.float32)
        return c

    lax.fori_loop(0, 27, zv, 0)

    def zt(i, c):
        pltpu.sync_copy(zb, agg.at[pl.ds(sid * NTR + i * 27, 27)])
        return c

    lax.fori_loop(0, NTR // 27, zt, 0)
    plsc.subcore_barrier()

    def batch(g, _):
        pltpu.async_copy(x1_hbm.at[srcb.at[g]], xbuf, sem).wait()

        def sub(sb, c):
            pltpu.async_copy(
                w2_hbm.at[tbf.at[pl.ds(g * 128 + sb * WB, WB)]], w2b,
                sem).wait()

            def edge(es, c2):
                e = sb * WB + es
                n = c0f[pl.ds(g * 128 + e, LL)][0]
                for cc in range(4):
                    hv = xbuf[e, pl.ds(cc * LL, LL)]
                    for o2 in range(2):
                        acc = hv[0] * w2b[es, pl.ds(cc * 512 + o2 * LL, LL)]
                        for i in range(1, 16):
                            acc = acc + hv[i] * w2b[
                                es, pl.ds(cc * 512 + i * 32 + o2 * LL, LL)]
                        msgb[e, pl.ds(cc * 32 + o2 * LL, LL)] = acc * n
                return c2

            lax.fori_loop(0, WB, edge, 0)
            return c

        lax.fori_loop(0, 128 // WB, sub, 0)
        pltpu.sync_copy(msgb, agg.at[dstb.at[g]], add=True)
        return 0

    lax.fori_loop(0, ROWS32, batch, 0)
    plsc.subcore_barrier()
    pltpu.sync_copy(agg.at[pl.ds(sid * NTR, NTR)],
                    agg_hbm.at[cid, pl.ds(sid * NTR, NTR)])


def _s3(x1p, w2f, srcr, dstr, tf, cntf):
    f = pl.kernel(
        _s3_body,
        out_type=jax.ShapeDtypeStruct((2, NNP, DD), jnp.float32),
        mesh=_mesh(),
        scratch_types=[
            pltpu.VMEM_SHARED((NNP, DD), jnp.float32),
            pltpu.VMEM((ROWS32, 128), jnp.int32),
            pltpu.VMEM((ROWS32, 128), jnp.int32),
            pltpu.VMEM((EP // NWK + LL,), jnp.int32),
            pltpu.VMEM((EP // NWK + LL,), jnp.float32),
            pltpu.VMEM((EP // NWK + LL,), jnp.float32),
            pltpu.VMEM((128, DD), jnp.float32),
            pltpu.VMEM((128, DD), jnp.float32),
            pltpu.VMEM((WB, 2048), jnp.float32),
            pltpu.VMEM((27, 128), jnp.float32),
            pltpu.SemaphoreType.DMA,
        ],
    )
    return f(x1p, w2f, srcr, dstr, tf, cntf)


def _counts_jnp(dstp, tp):
    key = dstp * NR + tp
    counts = jnp.zeros((4 * QS,), jnp.float32).at[key].add(1.0)
    c = counts[key]
    half = (key < 2 * QS).astype(jnp.float32)
    return jnp.stack([c * half, c * (1.0 - half)])


def _l1_edges_jnp(xb2, comp, srcp, dstp, tp, norm):
    c = comp[tp]
    msg = (c[:, :1] * xb2[srcp, :HID] + c[:, 1:] * xb2[srcp, HID:]) * norm[:, None]
    agg = jnp.zeros((2, NNP, HID), jnp.float32).at[0, dstp].add(msg)
    return agg


def _l2_edges_jnp(x1, W2f, srcp, dstp, tp, norm):
    x1r = x1.reshape(NN, 4, 16)
    W2 = W2f.reshape(NR, 4, 16, 32)
    msg = jnp.einsum('eci,ecio->eco', x1r[srcp], W2[tp]).reshape(EP, DD)
    msg = msg * norm[:, None]
    agg = jnp.zeros((2, NNP, DD), jnp.float32).at[0, dstp].add(msg)
    return agg


def _score_jnp(x2, rel, ei0p, ei1p, n0p, n1p, etp):
    rv = rel[etp]
    dp = x2[ei0p] + rv - x2[ei1p]
    dn = x2[n0p] + rv - x2[n1p]
    return jnp.sum(dp * dp, axis=1), jnp.sum(dn * dn, axis=1)


def _pad_e(a, fill):
    return jnp.concatenate(
        [a, jnp.full((EP - NE,), fill, a.dtype)], axis=0)


def kernel(emb, comp, bases, root1, bias1, W2, root2, bias2, rel,
           total_index, total_type, edge_index, edge_type):
    w1 = jnp.concatenate(
        [jnp.concatenate([bases[0], bases[1]], axis=1), root1], axis=1)
    srcp = _pad_e(total_index[0], 0)
    dstp = _pad_e(total_index[1], NN)
    tp = _pad_e(total_type, 0)

    xb2, y1 = _t1(emb, w1, bias1)

    cnt2 = _s1(dstp.reshape(ER, 128), tp.reshape(ER, 128))
    valid = (lax.iota(jnp.int32, EP) < NE).astype(jnp.float32)
    norm = valid / (cnt2[0].reshape(EP) + cnt2[1].reshape(EP))

    compf = jnp.pad(comp.reshape(2 * NR), (0, 2048 - 2 * NR))
    agg1p = _s2(xb2, compf, srcp.reshape(ER, 128), dstp.reshape(ER, 128),
                tp, cnt2.reshape(2, EP))
    x1, y2 = _t2(agg1p, y1, root2, bias2)

    agg2p = _s3(x1, W2.reshape(NR, 4 * 16 * 32),
                srcp.reshape(ER, 128), dstp.reshape(ER, 128),
                tp, cnt2.reshape(2, EP))
    x2 = _t3(agg2p, y2)

    neg_ei = jax.random.randint(jax.random.key(123), (2, NE), 0, NN)
    ei0p = _pad_e(edge_index[0], 0).reshape(ER, 128)
    ei1p = _pad_e(edge_index[1], 0).reshape(ER, 128)
    n0p = _pad_e(neg_ei[0], 0).reshape(ER, 128)
    n1p = _pad_e(neg_ei[1], 0).reshape(ER, 128)
    etp = _pad_e(edge_type, 0).reshape(ER, 128)
    dp, dn = _s4(x2, rel, ei0p, ei1p, n0p, n1p, etp)

    return _t4(dp, dn)
